# baseline (speedup 1.0000x reference)
import functools

import jax
import jax.numpy as jnp
from jax import lax
from jax.experimental import pallas as pl
from jax.experimental.pallas import tpu as pltpu
from jax.experimental.pallas import tpu_sc as plsc

_B = 16384
_D = 128
_L = 16
_NC, _NS = 2, 16
_NW = _NC * _NS
_BPW = _B // _NW
_IDXC = 128
_NIDX = _BPW // _IDXC
_SUB = 32
_NSUB = _BPW // _SUB
_NSLOT = 8
_EPS = 1e-8
_MAGIC = 0x5F3759DF


def _shuf(v, sh):
    dnums = lax.GatherDimensionNumbers(
        offset_dims=(), collapsed_slice_dims=(0,), start_index_map=(0,))
    perm = (lax.iota(jnp.int32, _L) ^ sh).reshape(_L, 1)
    return lax.gather(v, perm, dnums, (1,),
                      mode=lax.GatherScatterMode.PROMISE_IN_BOUNDS)


def _fold_merge(ta, tb, sh, mask):
    return tuple(jnp.where(mask, a + _shuf(a, sh), b + _shuf(b, sh))
                 for a, b in zip(ta, tb))


def _rsqrt(p):
    bits = lax.bitcast_convert_type(p, jnp.int32)
    y = lax.bitcast_convert_type(_MAGIC - (bits >> 1), jnp.float32)
    for _ in range(3):
        y = y * (1.5 - 0.5 * p * y * y)
    return y


def _sc_body(users_h, items_h, ut_h, it_h, out_h,
             idx_u, idx_i, ub, ib, outb, sem_idx, semu, semi, sem_out):
    wid = lax.axis_index("s") * _NC + lax.axis_index("c")
    base = wid * _BPW
    lane = lax.iota(jnp.int32, _L)
    masks = {sh: (lane & sh) == 0 for sh in (8, 4, 2, 1)}

    idx_copies = []
    for q in range(_NIDX):
        idx_copies.append(pltpu.make_async_copy(
            users_h.at[pl.ds(base + q * _IDXC, _IDXC)], idx_u.at[q],
            sem_idx.at[q]))
        idx_copies.append(pltpu.make_async_copy(
            items_h.at[pl.ds(base + q * _IDXC, _IDXC)], idx_i.at[q],
            sem_idx.at[q]))
    for h in idx_copies:
        h.start()

    def gather(c, slot):
        q = c // (_IDXC // _SUB)
        r = (c % (_IDXC // _SUB)) * _SUB
        sl = pl.ds(slot * _SUB, _SUB)
        return (pltpu.make_async_copy(
                    ut_h.at[idx_u.at[q, pl.ds(r, _SUB)]], ub.at[sl],
                    semu.at[slot]),
                pltpu.make_async_copy(
                    it_h.at[idx_i.at[q, pl.ds(r, _SUB)]], ib.at[sl],
                    semi.at[slot]))

    subs_per_idx = _IDXC // _SUB
    for c in range(_NSLOT):
        if c % subs_per_idx == 0:
            q = c // subs_per_idx
            idx_copies[2 * q].wait()
            idx_copies[2 * q + 1].wait()
        for h in gather(c, c):
            h.start()
    for q in range(_NSLOT // subs_per_idx, _NIDX):
        idx_copies[2 * q].wait()
        idx_copies[2 * q + 1].wait()

    def row_acc(r):
        an = jnp.zeros((_L,), jnp.float32)
        au = jnp.zeros((_L,), jnp.float32)
        ai = jnp.zeros((_L,), jnp.float32)
        for j in range(_D // _L):
            u = ub[r, pl.ds(j * _L, _L)]
            v = ib[r, pl.ds(j * _L, _L)]
            an = an + u * v
            au = au + u * u
            ai = ai + v * v
        return an, au, ai

    def chunk(c, _):
        slot = lax.rem(c, _NSLOT)
        rb = slot * _SUB
        for h in gather(c, slot):
            h.wait()

        @plsc.parallel_loop(0, _SUB // _L)
        def group(g):
            r0 = rb + g * _L
            pairs = [_fold_merge(row_acc(r0 + r), row_acc(r0 + r + 8),
                                 8, masks[8]) for r in range(8)]
            quads = [_fold_merge(pairs[r], pairs[r + 4], 4, masks[4])
                     for r in range(4)]
            duos = [_fold_merge(quads[r], quads[r + 2], 2, masks[2])
                    for r in range(2)]
            nv, au, ai = _fold_merge(duos[0], duos[1], 1, masks[1])
            pv = au * ai
            recip = jnp.where(pv >= _EPS * _EPS, _rsqrt(pv),
                              jnp.float32(1.0 / _EPS))
            outb[pl.ds(c * _SUB + g * _L, _L)] = nv * recip

        @pl.when(c + _NSLOT < _NSUB)
        def _():
            for h in gather(c + _NSLOT, slot):
                h.start()

        return 0

    lax.fori_loop(0, _NSUB, chunk, 0)
    out_copy = pltpu.make_async_copy(
        outb, out_h.at[pl.ds(base, _BPW)], sem_out)
    out_copy.start()
    out_copy.wait()


@jax.jit
def kernel(users, items, user_table, item_table):
    mesh = plsc.VectorSubcoreMesh(core_axis_name="c", subcore_axis_name="s")
    f = functools.partial(
        pl.kernel,
        mesh=mesh,
        out_type=jax.ShapeDtypeStruct((_B,), jnp.float32),
        scratch_types=[
            pltpu.VMEM((_NIDX, _IDXC), jnp.int32),
            pltpu.VMEM((_NIDX, _IDXC), jnp.int32),
            pltpu.VMEM((_NSLOT * _SUB, _D), jnp.float32),
            pltpu.VMEM((_NSLOT * _SUB, _D), jnp.float32),
            pltpu.VMEM((_BPW,), jnp.float32),
            pltpu.SemaphoreType.DMA((_NIDX,)),
            pltpu.SemaphoreType.DMA((_NSLOT,)),
            pltpu.SemaphoreType.DMA((_NSLOT,)),
            pltpu.SemaphoreType.DMA,
        ],
    )(_sc_body)
    return f(users, items, user_table, item_table)


# speedup vs baseline: 1.0041x; 1.0041x over previous
import functools

import jax
import jax.numpy as jnp
from jax import lax
from jax.experimental import pallas as pl
from jax.experimental.pallas import tpu as pltpu
from jax.experimental.pallas import tpu_sc as plsc

_B = 16384
_D = 128
_L = 16
_NC, _NS = 2, 16
_NW = _NC * _NS
_BPW = _B // _NW
_IDXC = 128
_NIDX = _BPW // _IDXC
_SUB = 32
_NSUB = _BPW // _SUB
_NSLOT = 8
_EPS = 1e-8
_MAGIC = 0x5F3759DF


def _shuf(v, sh):
    dnums = lax.GatherDimensionNumbers(
        offset_dims=(), collapsed_slice_dims=(0,), start_index_map=(0,))
    perm = (lax.iota(jnp.int32, _L) ^ sh).reshape(_L, 1)
    return lax.gather(v, perm, dnums, (1,),
                      mode=lax.GatherScatterMode.PROMISE_IN_BOUNDS)


def _fold_merge(ta, tb, sh, mask):
    return tuple(jnp.where(mask, a + _shuf(a, sh), b + _shuf(b, sh))
                 for a, b in zip(ta, tb))


def _rsqrt(p):
    bits = lax.bitcast_convert_type(p, jnp.int32)
    y = lax.bitcast_convert_type(_MAGIC - (bits >> 1), jnp.float32)
    for _ in range(3):
        y = y * (1.5 - 0.5 * p * y * y)
    return y


def _sc_body(users_h, items_h, ut_h, it_h, out_h,
             idx_u, idx_i, ub, ib, outb, sem_idx, semu, semi, sem_out):
    wid = lax.axis_index("s") * _NC + lax.axis_index("c")
    base = wid * _BPW
    lane = lax.iota(jnp.int32, _L)
    masks = {sh: (lane & sh) == 0 for sh in (8, 4, 2, 1)}

    idx_copies = []
    for q in range(_NIDX):
        idx_copies.append(pltpu.make_async_copy(
            users_h.at[pl.ds(base + q * _IDXC, _IDXC)], idx_u.at[q],
            sem_idx.at[q]))
        idx_copies.append(pltpu.make_async_copy(
            items_h.at[pl.ds(base + q * _IDXC, _IDXC)], idx_i.at[q],
            sem_idx.at[q]))
    for h in idx_copies:
        h.start()

    def gather(c, slot):
        q = c // (_IDXC // _SUB)
        r = (c % (_IDXC // _SUB)) * _SUB
        sl = pl.ds(slot * _SUB, _SUB)
        return (pltpu.make_async_copy(
                    ut_h.at[idx_u.at[q, pl.ds(r, _SUB)]], ub.at[sl],
                    semu.at[slot]),
                pltpu.make_async_copy(
                    it_h.at[idx_i.at[q, pl.ds(r, _SUB)]], ib.at[sl],
                    semi.at[slot]))

    subs_per_idx = _IDXC // _SUB
    for c in range(_NSLOT):
        if c % subs_per_idx == 0:
            q = c // subs_per_idx
            idx_copies[2 * q].wait()
            idx_copies[2 * q + 1].wait()
        for h in gather(c, c):
            h.start()
    for q in range(_NSLOT // subs_per_idx, _NIDX):
        idx_copies[2 * q].wait()
        idx_copies[2 * q + 1].wait()

    def row_acc(r):
        an = jnp.zeros((_L,), jnp.float32)
        au = jnp.zeros((_L,), jnp.float32)
        ai = jnp.zeros((_L,), jnp.float32)
        for j in range(_D // _L):
            u = ub[r, pl.ds(j * _L, _L)]
            v = ib[r, pl.ds(j * _L, _L)]
            an = an + u * v
            au = au + u * u
            ai = ai + v * v
        return an, au, ai

    def chunk(c, _):
        slot = lax.rem(c, _NSLOT)
        rb = slot * _SUB
        for h in gather(c, slot):
            h.wait()

        def group(g, _):
            r0 = rb + g * _L
            pairs = [_fold_merge(row_acc(r0 + r), row_acc(r0 + r + 8),
                                 8, masks[8]) for r in range(8)]
            quads = [_fold_merge(pairs[r], pairs[r + 4], 4, masks[4])
                     for r in range(4)]
            duos = [_fold_merge(quads[r], quads[r + 2], 2, masks[2])
                    for r in range(2)]
            nv, au, ai = _fold_merge(duos[0], duos[1], 1, masks[1])
            pv = au * ai
            recip = jnp.where(pv >= _EPS * _EPS, _rsqrt(pv),
                              jnp.float32(1.0 / _EPS))
            outb[pl.ds(c * _SUB + g * _L, _L)] = nv * recip
            return 0

        lax.fori_loop(0, _SUB // _L, group, 0)

        @pl.when(c + _NSLOT < _NSUB)
        def _():
            for h in gather(c + _NSLOT, slot):
                h.start()

        return 0

    lax.fori_loop(0, _NSUB, chunk, 0)
    out_copy = pltpu.make_async_copy(
        outb, out_h.at[pl.ds(base, _BPW)], sem_out)
    out_copy.start()
    out_copy.wait()


@jax.jit
def kernel(users, items, user_table, item_table):
    mesh = plsc.VectorSubcoreMesh(core_axis_name="c", subcore_axis_name="s")
    f = functools.partial(
        pl.kernel,
        mesh=mesh,
        out_type=jax.ShapeDtypeStruct((_B,), jnp.float32),
        scratch_types=[
            pltpu.VMEM((_NIDX, _IDXC), jnp.int32),
            pltpu.VMEM((_NIDX, _IDXC), jnp.int32),
            pltpu.VMEM((_NSLOT * _SUB, _D), jnp.float32),
            pltpu.VMEM((_NSLOT * _SUB, _D), jnp.float32),
            pltpu.VMEM((_BPW,), jnp.float32),
            pltpu.SemaphoreType.DMA((_NIDX,)),
            pltpu.SemaphoreType.DMA((_NSLOT,)),
            pltpu.SemaphoreType.DMA((_NSLOT,)),
            pltpu.SemaphoreType.DMA,
        ],
    )(_sc_body)
    return f(users, items, user_table, item_table)
